# baseline (speedup 1.0000x reference)
import functools

import jax
import jax.numpy as jnp
from jax import lax
from jax.experimental import pallas as pl
from jax.experimental.pallas import tpu as pltpu
from jax.experimental.pallas import tpu_sc as plsc

_L = 16
_TILE = 128
_NBUF = 12
_NHALF = 2


@functools.lru_cache(maxsize=None)
def _build(seq_len: int, vocab: int, dim: int):
    info = plsc.get_sparse_core_info()
    nc, ns = info.num_cores, info.num_subcores
    nw = nc * ns
    assert seq_len % (nw * _L * _NHALF) == 0
    bpw = seq_len // nw
    hpw = bpw // _NHALF
    ngroups = hpw // _L
    assert dim % _L == 0
    nr = dim // _L

    mesh = plsc.VectorSubcoreMesh(core_axis_name="c", subcore_axis_name="s")

    @functools.partial(
        pl.kernel,
        mesh=mesh,
        out_type=jax.ShapeDtypeStruct((dim, seq_len), jnp.float32),
        scratch_types=[
            pltpu.VMEM((bpw,), jnp.int32),
            pltpu.VMEM((_NBUF * dim, _TILE), jnp.float32),
            pltpu.VMEM((dim, hpw), jnp.float32),
            pltpu.VMEM((dim, hpw), jnp.float32),
            [pltpu.SemaphoreType.DMA] * _NBUF,
            pltpu.SemaphoreType.DMA,
        ],
        compiler_params=pltpu.CompilerParams(needs_layout_passes=False),
    )
    def emb(x_hbm, wt_hbm, pt_hbm, out_hbm, idx_v, ring_v, buf_v, pos_v, sems, psem):
        wid = lax.axis_index("s") * nc + lax.axis_index("c")
        base = wid * bpw

        pltpu.sync_copy(x_hbm.at[pl.ds(base, bpw)], idx_v)

        iota = lax.iota(jnp.int32, _L)
        nfire = min(_NBUF, _L)

        def fire(k, t128):
            tk = pl.multiple_of(t128[k], _TILE)
            b = k % _NBUF
            pltpu.async_copy(
                wt_hbm.at[:, pl.ds(tk, _TILE)],
                ring_v.at[pl.ds(b * dim, dim), :],
                sems[b],
            )

        def half(h):
            hbase = base + h * hpw
            pltpu.async_copy(pt_hbm.at[:, pl.ds(hbase, hpw)], pos_v, psem).wait()

            def group(gl):
                vec = idx_v[pl.ds(h * hpw + gl * _L, _L)]
                t128 = vec & jnp.int32(-_TILE)
                lanes = vec & jnp.int32(_TILE - 1)
                for k in range(nfire):
                    fire(k, t128)
                for k in range(_L):
                    b = k % _NBUF
                    pltpu.make_async_copy(
                        wt_hbm.at[:, pl.ds(0, _TILE)],
                        ring_v.at[pl.ds(b * dim, dim), :],
                        sems[b],
                    ).wait()
                    col = jnp.full((_L,), gl * _L + k, jnp.int32)
                    lane = jnp.full((_L,), lanes[k], jnp.int32)
                    for r in range(nr):
                        rows = iota + (b * dim + r * _L)
                        val = plsc.load_gather(ring_v, [rows, lane])
                        prow = iota + r * _L
                        pv = plsc.load_gather(pos_v, [prow, col])
                        plsc.store_scatter(buf_v, [prow, col], val + pv)
                    if k + _NBUF < _L:
                        fire(k + _NBUF, t128)

            pl.loop(0, ngroups)(group)
            pltpu.sync_copy(buf_v, out_hbm.at[:, pl.ds(hbase, hpw)])

        pl.loop(0, _NHALF)(half)

    return emb


def kernel(x, word_table, pos_table):
    seq_len = x.shape[0]
    vocab, dim = word_table.shape
    emb = _build(seq_len, vocab, dim)
    out_t = emb(x.astype(jnp.int32), word_table.T, pos_table[:seq_len].T)
    return out_t.T


# speedup vs baseline: 1.0169x; 1.0169x over previous
import functools

import jax
import jax.numpy as jnp
from jax import lax
from jax.experimental import pallas as pl
from jax.experimental.pallas import tpu as pltpu
from jax.experimental.pallas import tpu_sc as plsc

_L = 16
_TILE = 128
_NBUF = 8
_NHALF = 1


@functools.lru_cache(maxsize=None)
def _build(seq_len: int, vocab: int, dim: int):
    info = plsc.get_sparse_core_info()
    nc, ns = info.num_cores, info.num_subcores
    nw = nc * ns
    assert seq_len % (nw * _L * _NHALF) == 0
    bpw = seq_len // nw
    hpw = bpw // _NHALF
    ngroups = hpw // _L
    assert dim % _L == 0
    nr = dim // _L

    mesh = plsc.VectorSubcoreMesh(core_axis_name="c", subcore_axis_name="s")

    @functools.partial(
        pl.kernel,
        mesh=mesh,
        out_type=jax.ShapeDtypeStruct((dim, seq_len), jnp.float32),
        scratch_types=[
            pltpu.VMEM((bpw,), jnp.int32),
            pltpu.VMEM((_NBUF * dim, _TILE), jnp.float32),
            pltpu.VMEM((dim, hpw), jnp.float32),
            pltpu.VMEM((dim, hpw), jnp.float32),
            [pltpu.SemaphoreType.DMA] * _NBUF,
            pltpu.SemaphoreType.DMA,
        ],
        compiler_params=pltpu.CompilerParams(needs_layout_passes=False),
    )
    def emb(x_hbm, wt_hbm, pt_hbm, out_hbm, idx_v, ring_v, buf_v, pos_v, sems, psem):
        wid = lax.axis_index("s") * nc + lax.axis_index("c")
        base = wid * bpw

        pltpu.sync_copy(x_hbm.at[pl.ds(base, bpw)], idx_v)

        iota = lax.iota(jnp.int32, _L)
        nfire = min(_NBUF, _L)

        def fire(k, t128):
            tk = pl.multiple_of(t128[k], _TILE)
            b = k % _NBUF
            for r8 in range(dim // 8):
                pltpu.async_copy(
                    wt_hbm.at[pl.ds(8 * r8, 8), pl.ds(tk, _TILE)],
                    ring_v.at[pl.ds(b * dim + 8 * r8, 8), :],
                    sems[b],
                )

        def half(h):
            hbase = base + h * hpw
            pltpu.async_copy(pt_hbm.at[:, pl.ds(hbase, hpw)], pos_v, psem).wait()

            def group(gl):
                vec = idx_v[pl.ds(h * hpw + gl * _L, _L)]
                t128 = vec & jnp.int32(-_TILE)
                lanes = vec & jnp.int32(_TILE - 1)
                for k in range(nfire):
                    fire(k, t128)
                for k in range(_L):
                    b = k % _NBUF
                    for r8 in range(dim // 8):
                        pltpu.make_async_copy(
                            wt_hbm.at[pl.ds(0, 8), pl.ds(0, _TILE)],
                            ring_v.at[pl.ds(b * dim + 8 * r8, 8), :],
                            sems[b],
                        ).wait()
                    col = jnp.full((_L,), gl * _L + k, jnp.int32)
                    lane = jnp.full((_L,), lanes[k], jnp.int32)
                    for r in range(nr):
                        rows = iota + (b * dim + r * _L)
                        val = plsc.load_gather(ring_v, [rows, lane])
                        prow = iota + r * _L
                        pv = plsc.load_gather(pos_v, [prow, col])
                        plsc.store_scatter(buf_v, [prow, col], val + pv)
                    if k + _NBUF < _L:
                        fire(k + _NBUF, t128)

            pl.loop(0, ngroups)(group)
            pltpu.sync_copy(buf_v, out_hbm.at[:, pl.ds(hbase, hpw)])

        pl.loop(0, _NHALF)(half)

    return emb


def kernel(x, word_table, pos_table):
    seq_len = x.shape[0]
    vocab, dim = word_table.shape
    emb = _build(seq_len, vocab, dim)
    out_t = emb(x.astype(jnp.int32), word_table.T, pos_table[:seq_len].T)
    return out_t.T
